# baseline (speedup 1.0000x reference)
import numpy as np
import jax
import jax.numpy as jnp
from jax import lax
from jax.experimental import pallas as pl
from jax.experimental.pallas import tpu as pltpu
from jax.experimental.pallas import tpu_sc as plsc

ALPHA = 4
FB = 4
NC, NS = 2, 16
SPLIT = 4
NBUF = 3


def _copy_body(in_ref, fast_ref):
    fast_ref[...] = in_ref[...]


def _make_sc_gather(C, T, H, W, N, a, b):
    HH = H // SPLIT
    n_chunks = C * N * SPLIT
    n_workers = NC * NS
    per_w = n_chunks // n_workers
    assert n_chunks % n_workers == 0

    mesh = plsc.VectorSubcoreMesh(
        core_axis_name="c", subcore_axis_name="s",
        num_cores=NC, num_subcores=NS,
    )

    def body(frames_hbm, slow_hbm, *scratch):
        bufs = scratch[:NBUF]
        sin = scratch[NBUF:2 * NBUF]
        sout = scratch[2 * NBUF:]
        wid = lax.axis_index("s") * NC + lax.axis_index("c")

        def chunk(j):
            q = wid * per_w + j
            c = q // (N * SPLIT)
            r = q % (N * SPLIT)
            k = r // SPLIT
            part = r % SPLIT
            t = (k * a) // b
            h0 = part * HH
            return (
                frames_hbm.at[c, t, pl.ds(h0, HH), :],
                slow_hbm.at[c, k, pl.ds(h0, HH), :],
            )

        in_cp, out_cp = {}, {}
        for j in range(per_w):
            src, dst = chunk(j)
            if j >= NBUF:
                out_cp[j - NBUF].wait()
            in_cp[j] = pltpu.make_async_copy(src, bufs[j % NBUF], sin[j % NBUF])
            in_cp[j].start()
            if j >= 1:
                p = j - 1
                in_cp[p].wait()
                _, dstp = chunk(p)
                out_cp[p] = pltpu.make_async_copy(bufs[p % NBUF], dstp, sout[p % NBUF])
                out_cp[p].start()
        last = per_w - 1
        in_cp[last].wait()
        _, dstl = chunk(last)
        out_cp[last] = pltpu.make_async_copy(bufs[last % NBUF], dstl, sout[last % NBUF])
        out_cp[last].start()
        for j in range(max(0, per_w - NBUF), per_w):
            out_cp[j].wait()

    return pl.kernel(
        body,
        out_type=jax.ShapeDtypeStruct((C, N, H, W), jnp.float32),
        mesh=mesh,
        scratch_types=(
            [pltpu.VMEM((HH, W), jnp.float32) for _ in range(NBUF)]
            + [pltpu.SemaphoreType.DMA for _ in range(2 * NBUF)]
        ),
    )


def kernel(frames):
    C, T, H, W = frames.shape
    N = T // ALPHA
    a, b = T - 1, N - 1

    idx = np.linspace(0, T - 1, N).astype(np.int64)
    idx_arith = (np.arange(N) * a) // b
    assert np.array_equal(idx, idx_arith), (idx, idx_arith)

    slow = _make_sc_gather(C, T, H, W, N, a, b)(frames)

    fast = pl.pallas_call(
        _copy_body,
        grid=(T // FB,),
        in_specs=[pl.BlockSpec((C, FB, H, W), lambda s: (0, s, 0, 0))],
        out_specs=pl.BlockSpec((C, FB, H, W), lambda s: (0, s, 0, 0)),
        out_shape=jax.ShapeDtypeStruct((C, T, H, W), frames.dtype),
    )(frames)

    return (slow, fast)


# speedup vs baseline: 1.0191x; 1.0191x over previous
import numpy as np
import jax
import jax.numpy as jnp
from jax import lax
from jax.experimental import pallas as pl
from jax.experimental.pallas import tpu as pltpu
from jax.experimental.pallas import tpu_sc as plsc

ALPHA = 4
FB = 4
NC, NS = 1, 16
SPLIT = 4
NBUF = 3


def _copy_body(in_ref, fast_ref):
    fast_ref[...] = in_ref[...]


def _make_sc_gather(C, T, H, W, N, a, b):
    HH = H // SPLIT
    n_chunks = C * N * SPLIT
    n_workers = NC * NS
    per_w = n_chunks // n_workers
    assert n_chunks % n_workers == 0

    mesh = plsc.VectorSubcoreMesh(
        core_axis_name="c", subcore_axis_name="s",
        num_cores=NC, num_subcores=NS,
    )

    def body(frames_hbm, slow_hbm, *scratch):
        bufs = scratch[:NBUF]
        sin = scratch[NBUF:2 * NBUF]
        sout = scratch[2 * NBUF:]
        wid = lax.axis_index("s") * NC + lax.axis_index("c")

        def chunk(j):
            q = wid * per_w + j
            c = q // (N * SPLIT)
            r = q % (N * SPLIT)
            k = r // SPLIT
            part = r % SPLIT
            t = (k * a) // b
            h0 = part * HH
            return (
                frames_hbm.at[c, t, pl.ds(h0, HH), :],
                slow_hbm.at[c, k, pl.ds(h0, HH), :],
            )

        in_cp, out_cp = {}, {}
        for j in range(per_w):
            src, dst = chunk(j)
            if j >= NBUF:
                out_cp[j - NBUF].wait()
            in_cp[j] = pltpu.make_async_copy(src, bufs[j % NBUF], sin[j % NBUF])
            in_cp[j].start()
            if j >= 1:
                p = j - 1
                in_cp[p].wait()
                _, dstp = chunk(p)
                out_cp[p] = pltpu.make_async_copy(bufs[p % NBUF], dstp, sout[p % NBUF])
                out_cp[p].start()
        last = per_w - 1
        in_cp[last].wait()
        _, dstl = chunk(last)
        out_cp[last] = pltpu.make_async_copy(bufs[last % NBUF], dstl, sout[last % NBUF])
        out_cp[last].start()
        for j in range(max(0, per_w - NBUF), per_w):
            out_cp[j].wait()

    return pl.kernel(
        body,
        out_type=jax.ShapeDtypeStruct((C, N, H, W), jnp.float32),
        mesh=mesh,
        scratch_types=(
            [pltpu.VMEM((HH, W), jnp.float32) for _ in range(NBUF)]
            + [pltpu.SemaphoreType.DMA for _ in range(2 * NBUF)]
        ),
    )


def kernel(frames):
    C, T, H, W = frames.shape
    N = T // ALPHA
    a, b = T - 1, N - 1

    idx = np.linspace(0, T - 1, N).astype(np.int64)
    idx_arith = (np.arange(N) * a) // b
    assert np.array_equal(idx, idx_arith), (idx, idx_arith)

    slow = _make_sc_gather(C, T, H, W, N, a, b)(frames)

    fast = pl.pallas_call(
        _copy_body,
        grid=(T // FB,),
        in_specs=[pl.BlockSpec((C, FB, H, W), lambda s: (0, s, 0, 0))],
        out_specs=pl.BlockSpec((C, FB, H, W), lambda s: (0, s, 0, 0)),
        out_shape=jax.ShapeDtypeStruct((C, T, H, W), frames.dtype),
    )(frames)

    return (slow, fast)


# speedup vs baseline: 1.2469x; 1.2236x over previous
import numpy as np
import jax
import jax.numpy as jnp
from jax.experimental import pallas as pl
from jax.experimental.pallas import tpu as pltpu

ALPHA = 4


def _pack_body(in_ref, slow_hbm, fast_ref, sem, *, a, b):
    s = pl.program_id(0)
    loc = (s * a) // b - ALPHA * s
    slow_cp = pltpu.make_async_copy(
        in_ref.at[:, pl.ds(loc, 1)], slow_hbm.at[:, pl.ds(s, 1)], sem
    )
    slow_cp.start()
    fast_ref[...] = in_ref[...]
    slow_cp.wait()


def kernel(frames):
    C, T, H, W = frames.shape
    N = T // ALPHA
    a, b = T - 1, N - 1

    idx = np.linspace(0, T - 1, N).astype(np.int64)
    idx_arith = (np.arange(N) * a) // b
    assert np.array_equal(idx, idx_arith), (idx, idx_arith)
    assert np.array_equal(idx // ALPHA, np.arange(N)), idx

    slow, fast = pl.pallas_call(
        lambda i, so, fo, sem: _pack_body(i, so, fo, sem, a=a, b=b),
        grid=(N,),
        in_specs=[pl.BlockSpec((C, ALPHA, H, W), lambda s: (0, s, 0, 0))],
        out_specs=(
            pl.BlockSpec(memory_space=pltpu.MemorySpace.HBM),
            pl.BlockSpec((C, ALPHA, H, W), lambda s: (0, s, 0, 0)),
        ),
        out_shape=(
            jax.ShapeDtypeStruct((C, N, H, W), frames.dtype),
            jax.ShapeDtypeStruct((C, T, H, W), frames.dtype),
        ),
        scratch_shapes=[pltpu.SemaphoreType.DMA],
    )(frames)
    return (slow, fast)


# speedup vs baseline: 1.3227x; 1.0608x over previous
import numpy as np
import jax
import jax.numpy as jnp
from jax.experimental import pallas as pl

ALPHA = 4


def _pack_body(in_ref, slow_ref, fast_ref, *, a, b):
    s = pl.program_id(0)
    fast_ref[...] = in_ref[...]
    loc = (s * a) // b - ALPHA * s
    slow_ref[...] = in_ref[:, pl.ds(loc, 1)]


def kernel(frames):
    C, T, H, W = frames.shape
    N = T // ALPHA
    a, b = T - 1, N - 1

    idx = np.linspace(0, T - 1, N).astype(np.int64)
    idx_arith = (np.arange(N) * a) // b
    assert np.array_equal(idx, idx_arith), (idx, idx_arith)
    assert np.array_equal(idx // ALPHA, np.arange(N)), idx

    slow, fast = pl.pallas_call(
        lambda i, s, f: _pack_body(i, s, f, a=a, b=b),
        grid=(N,),
        in_specs=[pl.BlockSpec((C, ALPHA, H, W), lambda s: (0, s, 0, 0))],
        out_specs=(
            pl.BlockSpec((C, 1, H, W), lambda s: (0, s, 0, 0)),
            pl.BlockSpec((C, ALPHA, H, W), lambda s: (0, s, 0, 0)),
        ),
        out_shape=(
            jax.ShapeDtypeStruct((C, N, H, W), frames.dtype),
            jax.ShapeDtypeStruct((C, T, H, W), frames.dtype),
        ),
    )(frames)
    return (slow, fast)
